# baseline (speedup 1.0000x reference)
import functools

import jax
import jax.numpy as jnp
from jax import lax
from jax.experimental import pallas as pl
from jax.experimental.pallas import tpu as pltpu
from jax.experimental.pallas import tpu_sc as plsc

H = 100
W = 200
N = H * W
NUM_CLASSES = 80
MAX_OUT = 300
NP = 20480
R = NP // 128
NEG_INF = float("-inf")
BIG_F = 1.0e9

NT = 16
STRIPE = NP // NT
TV = STRIPE // 16
OUT_ROWS = 304


def _prep_body(cls_ref, ctr_ref, reg_ref, cen_ref, thr_ref,
               s_ref, x1_ref, y1_ref, x2_ref, y2_ref, area_ref, ids_ref):
    thr = thr_ref[0, 0]

    def cls_step(c, carry):
        acc, amax = carry
        x = cls_ref[c]
        gt = x > acc
        acc = jnp.where(gt, x, acc)
        amax = jnp.where(gt, c, amax)
        return acc, amax

    acc0 = cls_ref[0]
    amax0 = jnp.zeros((R, 128), jnp.int32)
    cls_scores, cls_ids = jax.lax.fori_loop(1, NUM_CLASSES, cls_step, (acc0, amax0))

    score = cls_scores * ctr_ref[...]
    fi = (jax.lax.broadcasted_iota(jnp.int32, (R, 128), 0) * 128
          + jax.lax.broadcasted_iota(jnp.int32, (R, 128), 1))
    s_ref[...] = jnp.where((score > thr) & (fi < N), score, NEG_INF)

    cx = cen_ref[0]
    cy = cen_ref[1]
    x1 = cx - reg_ref[0]
    y1 = cy - reg_ref[1]
    x2 = cx + reg_ref[2]
    y2 = cy + reg_ref[3]
    x1_ref[...] = x1
    y1_ref[...] = y1
    x2_ref[...] = x2
    y2_ref[...] = y2
    area_ref[...] = (x2 - x1) * (y2 - y1)
    ids_ref[...] = cls_ids.astype(jnp.float32)


def _nms_sc_body(s_hbm, x1_hbm, y1_hbm, x2_hbm, y2_hbm, area_hbm, ids_hbm,
                 par_hbm, out_hbm,
                 s_v, x1_v, y1_v, x2_v, y2_v, area_v, ids_v,
                 par_v, rec_v, all_rec_v, wrec_v, out_v, shared):
    cid = lax.axis_index("c")
    sid = lax.axis_index("s")
    base = sid * STRIPE

    pltpu.sync_copy(s_hbm.at[pl.ds(base, STRIPE)], s_v)
    pltpu.sync_copy(x1_hbm.at[pl.ds(base, STRIPE)], x1_v)
    pltpu.sync_copy(y1_hbm.at[pl.ds(base, STRIPE)], y1_v)
    pltpu.sync_copy(x2_hbm.at[pl.ds(base, STRIPE)], x2_v)
    pltpu.sync_copy(y2_hbm.at[pl.ds(base, STRIPE)], y2_v)
    pltpu.sync_copy(area_hbm.at[pl.ds(base, STRIPE)], area_v)
    pltpu.sync_copy(ids_hbm.at[pl.ds(base, STRIPE)], ids_v)
    pltpu.sync_copy(par_hbm, par_v)

    li = lax.iota(jnp.int32, 16)
    lif = li.astype(jnp.float32)
    basef = jnp.float32(1.0) * base
    iou_thr = par_v[...]
    zero16 = jnp.zeros((16,), jnp.int32)
    neginf16 = jnp.full((16,), NEG_INF, jnp.float32)

    acc = s_v[pl.ds(0, 16)]
    idxv = basef + lif
    for t in range(1, TV):
        sv = s_v[pl.ds(t * 16, 16)]
        gt = sv > acc
        acc = jnp.where(gt, sv, acc)
        idxv = jnp.where(gt, basef + (t * 16) + lif, idxv)

    def step(k, carry):
        acc, idxv = carry
        m = jnp.max(acc)
        idxf = jnp.min(jnp.where(acc == m, idxv, BIG_F))
        lidx = (idxf - basef).astype(jnp.int32)
        lidx16 = jnp.full((16,), 0, jnp.int32) + lidx
        gx1 = plsc.load_gather(x1_v, [lidx16])
        gy1 = plsc.load_gather(y1_v, [lidx16])
        gx2 = plsc.load_gather(x2_v, [lidx16])
        gy2 = plsc.load_gather(y2_v, [lidx16])
        gar = plsc.load_gather(area_v, [lidx16])
        gid = plsc.load_gather(ids_v, [lidx16])
        rec = jnp.where(li == 0, m,
              jnp.where(li == 1, idxf,
              jnp.where(li == 2, gx1,
              jnp.where(li == 3, gy1,
              jnp.where(li == 4, gx2,
              jnp.where(li == 5, gy2,
              jnp.where(li == 6, gid,
              jnp.where(li == 7, gar, 0.0))))))))
        rec_v[...] = rec

        par = lax.rem(k, 2)
        slot = par * (NT * 16) + sid * 16
        pltpu.sync_copy(rec_v, shared.at[pl.ds(slot, 16)])
        plsc.subcore_barrier()
        pltpu.sync_copy(shared.at[pl.ds(par * (NT * 16), NT * 16)], all_rec_v)

        scores = plsc.load_gather(all_rec_v, [li * 16 + 0])
        gidxs = plsc.load_gather(all_rec_v, [li * 16 + 1])
        wm = jnp.max(scores)
        widxf = jnp.min(jnp.where(scores == wm, gidxs, BIG_F))
        rrow = jnp.min(jnp.where((scores == wm) & (gidxs == widxf), li, 16))
        wrec = plsc.load_gather(all_rec_v, [zero16 + rrow * 16 + li])
        wrec_v[...] = wrec

        wx1 = plsc.load_gather(wrec_v, [zero16 + 2])
        wy1 = plsc.load_gather(wrec_v, [zero16 + 3])
        wx2 = plsc.load_gather(wrec_v, [zero16 + 4])
        wy2 = plsc.load_gather(wrec_v, [zero16 + 5])
        war = plsc.load_gather(wrec_v, [zero16 + 7])
        widx16 = jnp.full((16,), 0.0, jnp.float32) + widxf

        valid = wm > NEG_INF
        vf = jnp.where(valid, 1.0, 0.0)
        perm = jnp.where(li < 4, li + 2, jnp.where(li == 4, 0, jnp.where(li == 5, 6, 8)))
        og = plsc.load_gather(wrec_v, [perm])
        outrec = jnp.where(li <= 4, og * vf,
                 jnp.where(li == 5, jnp.where(valid, og, -1.0), 0.0))
        out_v[pl.ds(k * 16, 16)] = outrec

        nacc = neginf16
        nidxv = basef + lif
        for t in range(TV):
            sl = pl.ds(t * 16, 16)
            sv = s_v[sl]
            ix1 = jnp.maximum(wx1, x1_v[sl])
            iy1 = jnp.maximum(wy1, y1_v[sl])
            ix2 = jnp.minimum(wx2, x2_v[sl])
            iy2 = jnp.minimum(wy2, y2_v[sl])
            inter = jnp.maximum(ix2 - ix1, 0.0) * jnp.maximum(iy2 - iy1, 0.0)
            iou = inter / (war + area_v[sl] - inter + 1e-8)
            giv = basef + (t * 16) + lif
            supp = (iou > iou_thr) | (giv == widx16)
            sv = jnp.where(supp, NEG_INF, sv)
            s_v[sl] = sv
            if t == 0:
                nacc = sv
            else:
                gt = sv > nacc
                nacc = jnp.where(gt, sv, nacc)
                nidxv = jnp.where(gt, giv, nidxv)
        return nacc, nidxv

    lax.fori_loop(0, MAX_OUT, step, (acc, idxv), unroll=False)

    @pl.when(jnp.logical_and(cid == 0, sid == 0))
    def _():
        pltpu.sync_copy(out_v, out_hbm)


@jax.jit
def _decode_nms(cls_t, ctr_t, reg_t, centers, score_threshold, iou_threshold):
    pad = NP - N
    cls_p = jnp.pad(cls_t[0].T, ((0, 0), (0, pad))).reshape(NUM_CLASSES, R, 128)
    ctr_p = jnp.pad(ctr_t[0], ((0, pad),)).reshape(R, 128)
    reg_p = jnp.pad(reg_t[0].T, ((0, 0), (0, pad))).reshape(4, R, 128)
    cen_p = jnp.pad(centers.T, ((0, 0), (0, pad))).reshape(2, R, 128)
    thr = jnp.asarray(score_threshold, jnp.float32).reshape(1, 1)

    grid2d = jax.ShapeDtypeStruct((R, 128), jnp.float32)
    s0, x1, y1, x2, y2, area, idsf = pl.pallas_call(
        _prep_body,
        out_shape=[grid2d] * 7,
        in_specs=[
            pl.BlockSpec(memory_space=pltpu.VMEM),
            pl.BlockSpec(memory_space=pltpu.VMEM),
            pl.BlockSpec(memory_space=pltpu.VMEM),
            pl.BlockSpec(memory_space=pltpu.VMEM),
            pl.BlockSpec(memory_space=pltpu.SMEM),
        ],
        out_specs=[pl.BlockSpec(memory_space=pltpu.VMEM)] * 7,
    )(cls_p, ctr_p, reg_p, cen_p, thr)

    par = jnp.full((16,), jnp.asarray(iou_threshold, jnp.float32))

    nms = pl.kernel(
        _nms_sc_body,
        out_type=jax.ShapeDtypeStruct((OUT_ROWS * 16,), jnp.float32),
        mesh=plsc.VectorSubcoreMesh(core_axis_name="c", subcore_axis_name="s"),
        compiler_params=pltpu.CompilerParams(needs_layout_passes=False),
        scratch_types=[
            pltpu.VMEM((STRIPE,), jnp.float32),
            pltpu.VMEM((STRIPE,), jnp.float32),
            pltpu.VMEM((STRIPE,), jnp.float32),
            pltpu.VMEM((STRIPE,), jnp.float32),
            pltpu.VMEM((STRIPE,), jnp.float32),
            pltpu.VMEM((STRIPE,), jnp.float32),
            pltpu.VMEM((STRIPE,), jnp.float32),
            pltpu.VMEM((16,), jnp.float32),
            pltpu.VMEM((16,), jnp.float32),
            pltpu.VMEM((NT * 16,), jnp.float32),
            pltpu.VMEM((16,), jnp.float32),
            pltpu.VMEM((OUT_ROWS * 16,), jnp.float32),
            pltpu.VMEM_SHARED((2 * NT * 16,), jnp.float32),
        ],
    )

    out = nms(s0.reshape(NP), x1.reshape(NP), y1.reshape(NP),
              x2.reshape(NP), y2.reshape(NP), area.reshape(NP),
              idsf.reshape(NP), par)

    sel = out.reshape(OUT_ROWS, 16)[:MAX_OUT]
    out_boxes = sel[:, 0:4][None]
    out_scores = sel[:, 4][None]
    out_ids = sel[:, 5].astype(jnp.int32)[None]
    return out_boxes, out_scores, out_ids


def kernel(cls_target, ctr_target, reg_target, centers, score_threshold, iou_threshold):
    return _decode_nms(cls_target, ctr_target, reg_target, centers,
                       score_threshold, iou_threshold)
